# baseline (speedup 1.0000x reference)
import jax
import jax.numpy as jnp
from jax import lax
from jax.experimental import pallas as pl
from jax.experimental.pallas import tpu as pltpu
from jax.experimental.pallas import tpu_sc as plsc

_BATCH = 16
_SPATIAL = 16384
_FEAT = 64
_NSENS = 4096

_NC = 2
_NSUB = 16
_NW = _NC * _NSUB
_WPB = _NW // _BATCH
_SPW = _NSENS // _WPB
_SUB = 128
_NSUBCH = _SPW // _SUB
_L = 16


def _body(x_hbm, pos_hbm, w_hbm, out_hbm,
          idx_v, gidx_v, w_v, scale_v, g0, g1, o0, o1,
          gsem0, gsem1, wsem0, wsem1):
    wid = lax.axis_index("s") * _NC + lax.axis_index("c")
    b = wid // _WPB
    s_base = (wid % _WPB) * _SPW

    pltpu.sync_copy(pos_hbm.at[pl.ds(s_base, _SPW)], idx_v)
    pltpu.sync_copy(w_hbm, w_v)

    row_off = b * _SPATIAL

    def scale_body(g, carry):
        iv = idx_v[pl.ds(g * _L, _L)]
        wv = plsc.load_gather(w_v, [jnp.minimum(iv, _NSENS - 1)])
        scale_v[pl.ds(g * _L, _L)] = jnp.where(iv < _NSENS, wv, 1.0)
        gidx_v[g // (_SUB // _L), pl.ds((g % (_SUB // _L)) * _L, _L)] = iv + row_off
        return carry

    lax.fori_loop(0, _SPW // _L, scale_body, 0, unroll=4)

    gbufs = (g0, g1)
    obufs = (o0, o1)
    gsems = (gsem0, gsem1)
    wsems = (wsem0, wsem1)

    def start_gather(r, k):
        pltpu.async_copy(x_hbm.at[gidx_v.at[r]], gbufs[k], gsems[k])

    def wait_gather(k):
        pltpu.make_async_copy(x_hbm.at[gidx_v.at[0]], gbufs[k], gsems[k]).wait()

    def out_slab(r):
        return out_hbm.at[b, pl.ds(s_base + r * _SUB, _SUB)]

    def start_write(r, k):
        pltpu.async_copy(obufs[k], out_slab(r), wsems[k])

    def wait_write(k):
        pltpu.make_async_copy(obufs[k], out_slab(0), wsems[k]).wait()

    def scale_rows(r, k):
        gbuf, obuf = gbufs[k], obufs[k]

        def grp_body(grp, carry):
            vecs = scale_v[pl.ds(r * _SUB + grp * _L, _L)]
            for i in range(_L):
                vec = jnp.full((_L,), vecs[i], jnp.float32)
                row = grp * _L + i
                for j in range(_FEAT // _L):
                    obuf[row, pl.ds(j * _L, _L)] = (
                        gbuf[row, pl.ds(j * _L, _L)] * vec)
            return carry

        lax.fori_loop(0, _SUB // _L, grp_body, 0)

    start_gather(0, 0)
    start_gather(1, 1)

    def ring_body(gidx, carry):
        for k in range(2):
            r = gidx * 2 + k
            wait_gather(k)

            @pl.when(gidx > 0)
            def _():
                wait_write(k)

            scale_rows(r, k)

            @pl.when(r + 2 < _NSUBCH)
            def _():
                start_gather(r + 2, k)

            start_write(r, k)
        return carry

    lax.fori_loop(0, _NSUBCH // 2, ring_body, 0)
    wait_write(0)
    wait_write(1)


def kernel(x, sensor_positions, sensor_weights):
    xf = x.reshape(_BATCH * _SPATIAL, _FEAT)
    mesh = plsc.VectorSubcoreMesh(core_axis_name="c", subcore_axis_name="s")
    run = pl.kernel(
        _body,
        out_type=jax.ShapeDtypeStruct((_BATCH, _NSENS, _FEAT), jnp.float32),
        mesh=mesh,
        compiler_params=pltpu.CompilerParams(
            needs_layout_passes=False, use_tc_tiling_on_sc=False),
        scratch_types=[
            pltpu.VMEM((_SPW,), jnp.int32),
            pltpu.VMEM((_NSUBCH, _SUB), jnp.int32),
            pltpu.VMEM((_NSENS,), jnp.float32),
            pltpu.VMEM((_SPW,), jnp.float32),
            pltpu.VMEM((_SUB, _FEAT), jnp.float32),
            pltpu.VMEM((_SUB, _FEAT), jnp.float32),
            pltpu.VMEM((_SUB, _FEAT), jnp.float32),
            pltpu.VMEM((_SUB, _FEAT), jnp.float32),
            pltpu.SemaphoreType.DMA,
            pltpu.SemaphoreType.DMA,
            pltpu.SemaphoreType.DMA,
            pltpu.SemaphoreType.DMA,
        ],
    )
    return run(xf, sensor_positions.astype(jnp.int32), sensor_weights)


# speedup vs baseline: 1.3078x; 1.3078x over previous
import jax
import jax.numpy as jnp
from jax import lax
from jax.experimental import pallas as pl
from jax.experimental.pallas import tpu as pltpu
from jax.experimental.pallas import tpu_sc as plsc

_BATCH = 16
_SPATIAL = 16384
_FEAT = 64
_NSENS = 4096

_ROW = _BATCH * _FEAT
_NC = 2
_NSUB = 16
_NW = _NC * _NSUB
_SPW = _NSENS // _NW
_CH = 16
_NCHUNK = _SPW // _CH
_L = 16


def _body(xt_hbm, pos_hbm, w_hbm, out_hbm,
          idx_v, w_v, scale_v, g0, g1, o0, o1,
          gsem0, gsem1, wsem0, wsem1):
    wid = lax.axis_index("s") * _NC + lax.axis_index("c")
    s_base = wid * _SPW

    pltpu.sync_copy(pos_hbm.at[pl.ds(wid * _NCHUNK, _NCHUNK)], idx_v)
    pltpu.sync_copy(w_hbm, w_v)

    def scale_body(g, carry):
        iv = idx_v[g // (_CH // _L), pl.ds((g % (_CH // _L)) * _L, _L)]
        wv = plsc.load_gather(w_v, [jnp.minimum(iv, _NSENS - 1)])
        scale_v[pl.ds(g * _L, _L)] = jnp.where(iv < _NSENS, wv, 1.0)
        return carry

    lax.fori_loop(0, _SPW // _L, scale_body, 0, unroll=8)

    gbufs = (g0, g1)
    obufs = (o0, o1)
    gsems = (gsem0, gsem1)
    wsems = (wsem0, wsem1)

    def start_gather(c, k):
        pltpu.async_copy(xt_hbm.at[idx_v.at[c]], gbufs[k], gsems[k])

    def wait_gather(k):
        pltpu.make_async_copy(xt_hbm.at[idx_v.at[0]], gbufs[k], gsems[k]).wait()

    def out_slab(c):
        return out_hbm.at[pl.ds(s_base + c * _CH, _CH)]

    def start_write(c, k):
        pltpu.async_copy(obufs[k], out_slab(c), wsems[k])

    def wait_write(k):
        pltpu.make_async_copy(obufs[k], out_slab(0), wsems[k]).wait()

    def scale_chunk(c, k):
        gbuf, obuf = gbufs[k], obufs[k]

        def grp_body(grp, carry):
            vecs = scale_v[pl.ds(c * _CH + grp * _L, _L)]
            for i in range(_L):
                vec = jnp.full((_L,), vecs[i], jnp.float32)
                row = grp * _L + i
                for j in range(_ROW // _L):
                    obuf[row, pl.ds(j * _L, _L)] = (
                        gbuf[row, pl.ds(j * _L, _L)] * vec)
            return carry

        lax.fori_loop(0, _CH // _L, grp_body, 0)

    start_gather(0, 0)
    start_gather(1, 1)

    def ring_body(g, carry):
        for k in range(2):
            c = g * 2 + k
            wait_gather(k)

            @pl.when(g > 0)
            def _():
                wait_write(k)

            scale_chunk(c, k)

            @pl.when(c + 2 < _NCHUNK)
            def _():
                start_gather(c + 2, k)

            start_write(c, k)
        return carry

    lax.fori_loop(0, _NCHUNK // 2, ring_body, 0)
    wait_write(0)
    wait_write(1)


def kernel(x, sensor_positions, sensor_weights):
    xt = jnp.transpose(x, (1, 0, 2)).reshape(_SPATIAL, _ROW)
    mesh = plsc.VectorSubcoreMesh(core_axis_name="c", subcore_axis_name="s")
    run = pl.kernel(
        _body,
        out_type=jax.ShapeDtypeStruct((_NSENS, _ROW), jnp.float32),
        mesh=mesh,
        compiler_params=pltpu.CompilerParams(
            needs_layout_passes=False, use_tc_tiling_on_sc=False),
        scratch_types=[
            pltpu.VMEM((_NCHUNK, _CH), jnp.int32),
            pltpu.VMEM((_NSENS,), jnp.float32),
            pltpu.VMEM((_SPW,), jnp.float32),
            pltpu.VMEM((_CH, _ROW), jnp.float32),
            pltpu.VMEM((_CH, _ROW), jnp.float32),
            pltpu.VMEM((_CH, _ROW), jnp.float32),
            pltpu.VMEM((_CH, _ROW), jnp.float32),
            pltpu.SemaphoreType.DMA,
            pltpu.SemaphoreType.DMA,
            pltpu.SemaphoreType.DMA,
            pltpu.SemaphoreType.DMA,
        ],
    )
    pos2 = sensor_positions.astype(jnp.int32).reshape(_NSENS // _CH, _CH)
    yt = run(xt, pos2, sensor_weights)
    return jnp.transpose(yt.reshape(_NSENS, _BATCH, _FEAT), (1, 0, 2))


# speedup vs baseline: 1.9962x; 1.5263x over previous
import jax
import jax.numpy as jnp
from jax import lax
from jax.experimental import pallas as pl
from jax.experimental.pallas import tpu as pltpu
from jax.experimental.pallas import tpu_sc as plsc

_BATCH = 16
_SPATIAL = 16384
_FEAT = 64
_NSENS = 4096

_ROW = _BATCH * _FEAT
_NC = 2
_NSUB = 16
_NW = _NC * _NSUB
_SPW = _NSENS // _NW
_CH = 16
_NCHUNK = _SPW // _CH
_L = 16


def _body(xt_hbm, pos_hbm, w_hbm, out_hbm,
          idx_v, w_v, scale_v, g0, g1, o0, o1,
          gsem0, gsem1, wsem0, wsem1):
    wid = lax.axis_index("s") * _NC + lax.axis_index("c")
    s_base = wid * _SPW

    pltpu.sync_copy(pos_hbm.at[pl.ds(wid * _NCHUNK, _NCHUNK)], idx_v)
    pltpu.sync_copy(w_hbm, w_v)

    def scale_body(g, carry):
        iv = idx_v[g // (_CH // _L), pl.ds((g % (_CH // _L)) * _L, _L)]
        wv = plsc.load_gather(w_v, [jnp.minimum(iv, _NSENS - 1)])
        scale_v[pl.ds(g * _L, _L)] = jnp.where(iv < _NSENS, wv, 1.0)
        return carry

    lax.fori_loop(0, _SPW // _L, scale_body, 0, unroll=8)

    gbufs = (g0, g1)
    obufs = (o0, o1)
    gsems = (gsem0, gsem1)
    wsems = (wsem0, wsem1)

    def start_gather(c, k):
        pltpu.async_copy(xt_hbm.at[idx_v.at[c]], gbufs[k], gsems[k])

    def wait_gather(k):
        pltpu.make_async_copy(xt_hbm.at[idx_v.at[0]], gbufs[k], gsems[k]).wait()

    def out_slab(c):
        return out_hbm.at[pl.ds(s_base + c * _CH, _CH)]

    def start_write(c, k):
        pltpu.async_copy(obufs[k], out_slab(c), wsems[k])

    def wait_write(k):
        pltpu.make_async_copy(obufs[k], out_slab(0), wsems[k]).wait()

    def scale_chunk(c, k):
        gbuf, obuf = gbufs[k], obufs[k]

        def grp_body(grp, carry):
            vecs = scale_v[pl.ds(c * _CH + grp * _L, _L)]
            for i in range(_L):
                vec = jnp.full((_L,), vecs[i], jnp.float32)
                row = grp * _L + i
                for j in range(_ROW // _L):
                    obuf[row, pl.ds(j * _L, _L)] = (
                        gbuf[row, pl.ds(j * _L, _L)] * vec)
            return carry

        lax.fori_loop(0, _CH // _L, grp_body, 0)

    start_gather(0, 0)
    start_gather(1, 1)

    def ring_body(g, carry):
        for k in range(2):
            c = g * 2 + k
            wait_gather(k)

            @pl.when(g > 0)
            def _():
                wait_write(k)

            scale_chunk(c, k)

            @pl.when(c + 2 < _NCHUNK)
            def _():
                start_gather(c + 2, k)

            start_write(c, k)
        return carry

    lax.fori_loop(0, _NCHUNK // 2, ring_body, 0)
    wait_write(0)
    wait_write(1)


def kernel(x, sensor_positions, sensor_weights):
    xt = jnp.transpose(x, (1, 0, 2)).reshape(_SPATIAL, _ROW)
    mesh = plsc.VectorSubcoreMesh(core_axis_name="c", subcore_axis_name="s")
    run = pl.kernel(
        _body,
        out_type=jax.ShapeDtypeStruct((_NSENS, _ROW), jnp.float32),
        mesh=mesh,
        compiler_params=pltpu.CompilerParams(
            needs_layout_passes=False, use_tc_tiling_on_sc=True),
        scratch_types=[
            pltpu.VMEM((_NCHUNK, _CH), jnp.int32),
            pltpu.VMEM((_NSENS,), jnp.float32),
            pltpu.VMEM((_SPW,), jnp.float32),
            pltpu.VMEM((_CH, _ROW), jnp.float32),
            pltpu.VMEM((_CH, _ROW), jnp.float32),
            pltpu.VMEM((_CH, _ROW), jnp.float32),
            pltpu.VMEM((_CH, _ROW), jnp.float32),
            pltpu.SemaphoreType.DMA,
            pltpu.SemaphoreType.DMA,
            pltpu.SemaphoreType.DMA,
            pltpu.SemaphoreType.DMA,
        ],
    )
    pos2 = sensor_positions.astype(jnp.int32).reshape(_NSENS // _CH, _CH)
    yt = run(xt, pos2, sensor_weights)
    return jnp.transpose(yt.reshape(_NSENS, _BATCH, _FEAT), (1, 0, 2))
